# Initial kernel scaffold; baseline (speedup 1.0000x reference)
import jax
import jax.numpy as jnp
from jax.experimental import pallas as pl


def kernel(x, edge_index, edge_attr, W1, b1, W2, b2):
    raise NotImplementedError("write your pallas kernel here")


# speedup vs baseline: 3.7179x; 3.7179x over previous
import functools

import jax
import jax.numpy as jnp
from jax import lax
from jax.experimental import pallas as pl
from jax.experimental.pallas import tpu as pltpu
from jax.experimental.pallas import tpu_sc as plsc

N = 10000
E = 320000
D = 128
DE = 16

NC = 2
NS = 16
NW = NC * NS
EPW = E // NW
CHUNK = 80
NCHUNK = EPW // CHUNK
RPT = N // NS

BN = 2000
BE = 8000


def _pq_body(x_ref, wa_ref, wb_ref, b1_ref, p_ref, q_ref):
    xb = x_ref[...]
    p_ref[...] = jnp.dot(xb, wa_ref[...], preferred_element_type=jnp.float32)
    q_ref[...] = (jnp.dot(xb, wb_ref[...], preferred_element_type=jnp.float32)
                  + b1_ref[...])


def _prep_pq(x, w1a, w1b, b1):
    return pl.pallas_call(
        _pq_body,
        grid=(N // BN,),
        in_specs=[
            pl.BlockSpec((BN, D), lambda i: (i, 0)),
            pl.BlockSpec((D, D), lambda i: (0, 0)),
            pl.BlockSpec((D, D), lambda i: (0, 0)),
            pl.BlockSpec((1, D), lambda i: (0, 0)),
        ],
        out_specs=[
            pl.BlockSpec((BN, D), lambda i: (i, 0)),
            pl.BlockSpec((BN, D), lambda i: (i, 0)),
        ],
        out_shape=[
            jax.ShapeDtypeStruct((N, D), jnp.float32),
            jax.ShapeDtypeStruct((N, D), jnp.float32),
        ],
    )(x, w1a, w1b, b1.reshape(1, D))


def _a_body(ea_ref, wc_ref, a_ref):
    a_ref[...] = jnp.dot(ea_ref[...], wc_ref[...],
                         preferred_element_type=jnp.float32)


def _prep_a(edge_attr, w1c):
    return pl.pallas_call(
        _a_body,
        grid=(E // BE,),
        in_specs=[
            pl.BlockSpec((BE, DE), lambda i: (i, 0)),
            pl.BlockSpec((DE, D), lambda i: (0, 0)),
        ],
        out_specs=pl.BlockSpec((BE, D), lambda i: (i, 0)),
        out_shape=jax.ShapeDtypeStruct((E, D), jnp.float32),
    )(edge_attr, w1c)


def _sc_body(p_hbm, q_hbm, a_hbm, src_hbm, dst_hbm, s_out, c_out,
             s_sh, c_sh, src_v, dst_v, pbuf, qbuf, abuf, ones_v, z16,
             sem_p, sem_q, sem_a):
    core = lax.axis_index("c")
    sub = lax.axis_index("s")
    wid = core * NS + sub

    zero16 = jnp.zeros((16,), jnp.float32)

    @pl.loop(0, CHUNK)
    def _init(r):
        for j in range(D // 16):
            pbuf[r, pl.ds(j * 16, 16)] = zero16
        z16[r, :] = zero16
        ones_v[r, :] = zero16 + 1.0

    base = sub * RPT
    for k in range(RPT // CHUNK):
        pltpu.sync_copy(pbuf, s_sh.at[pl.ds(base + k * CHUNK, CHUNK)])
        pltpu.sync_copy(z16, c_sh.at[pl.ds(base + k * CHUNK, CHUNK)])
    rem = RPT % CHUNK
    pltpu.sync_copy(pbuf.at[pl.ds(0, rem)],
                    s_sh.at[pl.ds(base + RPT - rem, rem)])
    pltpu.sync_copy(z16.at[pl.ds(0, rem)],
                    c_sh.at[pl.ds(base + RPT - rem, rem)])

    plsc.subcore_barrier()

    ebase = wid * EPW

    @pl.loop(0, NCHUNK)
    def _chunk(g):
        off = ebase + g * CHUNK
        pltpu.sync_copy(src_hbm.at[pl.ds(off, CHUNK)], src_v)
        pltpu.sync_copy(dst_hbm.at[pl.ds(off, CHUNK)], dst_v)
        cp = pltpu.async_copy(p_hbm.at[src_v], pbuf, sem_p)
        cq = pltpu.async_copy(q_hbm.at[dst_v], qbuf, sem_q)
        ca = pltpu.async_copy(a_hbm.at[pl.ds(off, CHUNK)], abuf, sem_a)
        cp.wait()
        cq.wait()
        ca.wait()

        @pl.loop(0, CHUNK)
        def _row(r):
            for j in range(D // 16):
                sl = pl.ds(j * 16, 16)
                abuf[r, sl] = jnp.maximum(
                    pbuf[r, sl] + qbuf[r, sl] + abuf[r, sl], 0.0)

        pltpu.sync_copy(abuf, s_sh.at[dst_v], add=True)
        pltpu.sync_copy(ones_v, c_sh.at[dst_v], add=True)

    plsc.subcore_barrier()

    pltpu.sync_copy(s_sh.at[pl.ds(base, RPT)],
                    s_out.at[core, pl.ds(base, RPT)])
    pltpu.sync_copy(c_sh.at[pl.ds(base, RPT)],
                    c_out.at[core, pl.ds(base, RPT)])


def _sc_scatter(p, q, a, src, dst):
    mesh = plsc.VectorSubcoreMesh(core_axis_name="c", subcore_axis_name="s",
                                  num_cores=NC, num_subcores=NS)
    f = pl.kernel(
        _sc_body,
        out_type=(
            jax.ShapeDtypeStruct((NC, N, D), jnp.float32),
            jax.ShapeDtypeStruct((NC, N, DE), jnp.float32),
        ),
        mesh=mesh,
        scratch_types=[
            pltpu.VMEM_SHARED((N, D), jnp.float32),
            pltpu.VMEM_SHARED((N, DE), jnp.float32),
            pltpu.VMEM((CHUNK,), jnp.int32),
            pltpu.VMEM((CHUNK,), jnp.int32),
            pltpu.VMEM((CHUNK, D), jnp.float32),
            pltpu.VMEM((CHUNK, D), jnp.float32),
            pltpu.VMEM((CHUNK, D), jnp.float32),
            pltpu.VMEM((CHUNK, DE), jnp.float32),
            pltpu.VMEM((CHUNK, DE), jnp.float32),
            pltpu.SemaphoreType.DMA,
            pltpu.SemaphoreType.DMA,
            pltpu.SemaphoreType.DMA,
        ],
        compiler_params=pltpu.CompilerParams(use_tc_tiling_on_sc=False),
    )
    return f(p, q, a, src, dst)


def _final_body(x_ref, s_ref, c_ref, w2_ref, b2_ref, o_ref):
    sblk = s_ref[0] + s_ref[1]
    cnt = c_ref[0, :, :1] + c_ref[1, :, :1]
    o_ref[...] = (x_ref[...]
                  + jnp.dot(sblk, w2_ref[...],
                            preferred_element_type=jnp.float32)
                  + cnt * b2_ref[...])


def _final(x, s_part, c_part, w2, b2):
    return pl.pallas_call(
        _final_body,
        grid=(N // BN,),
        in_specs=[
            pl.BlockSpec((BN, D), lambda i: (i, 0)),
            pl.BlockSpec((NC, BN, D), lambda i: (0, i, 0)),
            pl.BlockSpec((NC, BN, DE), lambda i: (0, i, 0)),
            pl.BlockSpec((D, D), lambda i: (0, 0)),
            pl.BlockSpec((1, D), lambda i: (0, 0)),
        ],
        out_specs=pl.BlockSpec((BN, D), lambda i: (i, 0)),
        out_shape=jax.ShapeDtypeStruct((N, D), jnp.float32),
    )(x, s_part, c_part, w2, b2.reshape(1, D))


def kernel(x, edge_index, edge_attr, W1, b1, W2, b2):
    w1a = W1[:D]
    w1b = W1[D:2 * D]
    w1c = W1[2 * D:]
    src = edge_index[0]
    dst = edge_index[1]
    p, q = _prep_pq(x, w1a, w1b, b1)
    a = _prep_a(edge_attr, w1c)
    s_part, c_part = _sc_scatter(p, q, a, src, dst)
    return _final(x, s_part, c_part, W2, b2)


# speedup vs baseline: 5.4813x; 1.4743x over previous
import functools

import jax
import jax.numpy as jnp
from jax import lax
from jax.experimental import pallas as pl
from jax.experimental.pallas import tpu as pltpu
from jax.experimental.pallas import tpu_sc as plsc

N = 10000
E = 320000
D = 128
DE = 16

NC = 2
NS = 16
NW = NC * NS
EPW = E // NW
CHUNK = 40
NCHUNK = EPW // CHUNK
CPB = 50
NBLK = NCHUNK // CPB
RPT = N // NS

BN = 2000
BE = 8000


def _pq_body(x_ref, wa_ref, wb_ref, b1_ref, p_ref, q_ref):
    xb = x_ref[...]
    p_ref[...] = jnp.dot(xb, wa_ref[...], preferred_element_type=jnp.float32)
    q_ref[...] = (jnp.dot(xb, wb_ref[...], preferred_element_type=jnp.float32)
                  + b1_ref[...])


def _prep_pq(x, w1a, w1b, b1):
    return pl.pallas_call(
        _pq_body,
        grid=(N // BN,),
        in_specs=[
            pl.BlockSpec((BN, D), lambda i: (i, 0)),
            pl.BlockSpec((D, D), lambda i: (0, 0)),
            pl.BlockSpec((D, D), lambda i: (0, 0)),
            pl.BlockSpec((1, D), lambda i: (0, 0)),
        ],
        out_specs=[
            pl.BlockSpec((BN, D), lambda i: (i, 0)),
            pl.BlockSpec((BN, D), lambda i: (i, 0)),
        ],
        out_shape=[
            jax.ShapeDtypeStruct((N, D), jnp.float32),
            jax.ShapeDtypeStruct((N, D), jnp.float32),
        ],
    )(x, w1a, w1b, b1.reshape(1, D))


def _a_body(ea_ref, wc_ref, a_ref):
    a_ref[...] = jnp.dot(ea_ref[...], wc_ref[...],
                         preferred_element_type=jnp.float32)


def _prep_a(edge_attr, w1c):
    return pl.pallas_call(
        _a_body,
        grid=(E // BE,),
        in_specs=[
            pl.BlockSpec((BE, DE), lambda i: (i, 0)),
            pl.BlockSpec((DE, D), lambda i: (0, 0)),
        ],
        out_specs=pl.BlockSpec((BE, D), lambda i: (i, 0)),
        out_shape=jax.ShapeDtypeStruct((E, D), jnp.float32),
    )(edge_attr, w1c)


def _sc_body(p_hbm, q_hbm, a_hbm, src_hbm, dst_hbm, s_out, c_out,
             s_sh, c_sh, src_blk, dst_blk,
             pbuf, qbuf, abuf, ones_v, z16,
             sem_g0, sem_g1, sem_s0, sem_s1):
    core = lax.axis_index("c")
    sub = lax.axis_index("s")
    wid = core * NS + sub

    sem_g = (sem_g0, sem_g1)
    sem_s = (sem_s0, sem_s1)

    zero16 = jnp.zeros((16,), jnp.float32)

    @pl.loop(0, CHUNK)
    def _init(r):
        for j in range(D // 16):
            pbuf[0, r, pl.ds(j * 16, 16)] = zero16
        z16[r, :] = zero16
        ones_v[r, :] = zero16 + 1.0

    base = sub * RPT
    for k in range(RPT // CHUNK):
        pltpu.sync_copy(pbuf.at[0], s_sh.at[pl.ds(base + k * CHUNK, CHUNK)])
        pltpu.sync_copy(z16, c_sh.at[pl.ds(base + k * CHUNK, CHUNK)])
    rem = RPT % CHUNK
    pltpu.sync_copy(pbuf.at[0, pl.ds(0, rem)],
                    s_sh.at[pl.ds(base + RPT - rem, rem)])
    pltpu.sync_copy(z16.at[pl.ds(0, rem)],
                    c_sh.at[pl.ds(base + RPT - rem, rem)])

    plsc.subcore_barrier()

    ebase = wid * EPW

    def issue(bk, j, b):
        g = bk * CPB + j
        pltpu.async_copy(p_hbm.at[src_blk.at[j]], pbuf.at[b], sem_g[b])
        pltpu.async_copy(q_hbm.at[dst_blk.at[j]], qbuf.at[b], sem_g[b])
        pltpu.async_copy(a_hbm.at[pl.ds(ebase + g * CHUNK, CHUNK)],
                         abuf.at[b], sem_g[b])

    def wait_gathers(b):
        pltpu.make_async_copy(p_hbm.at[src_blk.at[0]], pbuf.at[b],
                              sem_g[b]).wait()
        pltpu.make_async_copy(q_hbm.at[dst_blk.at[0]], qbuf.at[b],
                              sem_g[b]).wait()
        pltpu.make_async_copy(a_hbm.at[pl.ds(0, CHUNK)], abuf.at[b],
                              sem_g[b]).wait()

    def compute(b):
        @pl.loop(0, CHUNK)
        def _row(r):
            for j in range(D // 16):
                sl = pl.ds(j * 16, 16)
                abuf[b, r, sl] = jnp.maximum(
                    pbuf[b, r, sl] + qbuf[b, r, sl] + abuf[b, r, sl], 0.0)

    def scatter(j, b):
        pltpu.async_copy(abuf.at[b], s_sh.at[dst_blk.at[j]], sem_s[b],
                         add=True)
        pltpu.async_copy(ones_v, c_sh.at[dst_blk.at[j]], sem_s[b], add=True)

    def wait_scatters(b):
        pltpu.make_async_copy(abuf.at[b], s_sh.at[dst_blk.at[0]],
                              sem_s[b]).wait()
        pltpu.make_async_copy(ones_v, c_sh.at[dst_blk.at[0]],
                              sem_s[b]).wait()

    @pl.loop(0, NBLK)
    def _blk(bk):
        pltpu.sync_copy(src_hbm.at[wid, pl.ds(bk * CPB, CPB)], src_blk)
        pltpu.sync_copy(dst_hbm.at[wid, pl.ds(bk * CPB, CPB)], dst_blk)
        issue(bk, 0, 0)

        @pl.loop(0, CPB // 2)
        def _pair(i):
            j0 = 2 * i

            @pl.when(i > 0)
            def _():
                wait_scatters(1)

            issue(bk, j0 + 1, 1)
            wait_gathers(0)
            compute(0)
            scatter(j0, 0)
            wait_scatters(0)

            @pl.when(i < CPB // 2 - 1)
            def _():
                issue(bk, j0 + 2, 0)

            wait_gathers(1)
            compute(1)
            scatter(j0 + 1, 1)

        wait_scatters(1)

    plsc.subcore_barrier()

    pltpu.sync_copy(s_sh.at[pl.ds(base, RPT)],
                    s_out.at[core, pl.ds(base, RPT)])
    pltpu.sync_copy(c_sh.at[pl.ds(base, RPT)],
                    c_out.at[core, pl.ds(base, RPT)])


def _sc_scatter(p, q, a, src, dst):
    mesh = plsc.VectorSubcoreMesh(core_axis_name="c", subcore_axis_name="s",
                                  num_cores=NC, num_subcores=NS)
    f = pl.kernel(
        _sc_body,
        out_type=(
            jax.ShapeDtypeStruct((NC, N, D), jnp.float32),
            jax.ShapeDtypeStruct((NC, N, DE), jnp.float32),
        ),
        mesh=mesh,
        scratch_types=[
            pltpu.VMEM_SHARED((N, D), jnp.float32),
            pltpu.VMEM_SHARED((N, DE), jnp.float32),
            pltpu.VMEM((CPB, CHUNK), jnp.int32),
            pltpu.VMEM((CPB, CHUNK), jnp.int32),
            pltpu.VMEM((2, CHUNK, D), jnp.float32),
            pltpu.VMEM((2, CHUNK, D), jnp.float32),
            pltpu.VMEM((2, CHUNK, D), jnp.float32),
            pltpu.VMEM((CHUNK, DE), jnp.float32),
            pltpu.VMEM((CHUNK, DE), jnp.float32),
            pltpu.SemaphoreType.DMA,
            pltpu.SemaphoreType.DMA,
            pltpu.SemaphoreType.DMA,
            pltpu.SemaphoreType.DMA,
        ],
        compiler_params=pltpu.CompilerParams(use_tc_tiling_on_sc=False),
    )
    return f(p, q, a, src.reshape(NW, NCHUNK, CHUNK),
             dst.reshape(NW, NCHUNK, CHUNK))


def _final_body(x_ref, s_ref, c_ref, w2_ref, b2_ref, o_ref):
    sblk = s_ref[0] + s_ref[1]
    cnt = c_ref[0, :, :1] + c_ref[1, :, :1]
    o_ref[...] = (x_ref[...]
                  + jnp.dot(sblk, w2_ref[...],
                            preferred_element_type=jnp.float32)
                  + cnt * b2_ref[...])


def _final(x, s_part, c_part, w2, b2):
    return pl.pallas_call(
        _final_body,
        grid=(N // BN,),
        in_specs=[
            pl.BlockSpec((BN, D), lambda i: (i, 0)),
            pl.BlockSpec((NC, BN, D), lambda i: (0, i, 0)),
            pl.BlockSpec((NC, BN, DE), lambda i: (0, i, 0)),
            pl.BlockSpec((D, D), lambda i: (0, 0)),
            pl.BlockSpec((1, D), lambda i: (0, 0)),
        ],
        out_specs=pl.BlockSpec((BN, D), lambda i: (i, 0)),
        out_shape=jax.ShapeDtypeStruct((N, D), jnp.float32),
    )(x, s_part, c_part, w2, b2.reshape(1, D))


def kernel(x, edge_index, edge_attr, W1, b1, W2, b2):
    w1a = W1[:D]
    w1b = W1[D:2 * D]
    w1c = W1[2 * D:]
    src = edge_index[0]
    dst = edge_index[1]
    p, q = _prep_pq(x, w1a, w1b, b1)
    a = _prep_a(edge_attr, w1c)
    s_part, c_part = _sc_scatter(p, q, a, src, dst)
    return _final(x, s_part, c_part, W2, b2)
